# baseline (speedup 1.0000x reference)
import functools

import jax
import jax.numpy as jnp
from jax import lax
from jax.experimental import pallas as pl
from jax.experimental.pallas import tpu as pltpu
from jax.experimental.pallas import tpu_sc as plsc

NC = 2
NS = 16
NW = NC * NS


def _deg_body(n, nblk, blk, ch, col_hbm, zn_hbm, out_hbm, call, obuf, deg_sh,
              sem):
    c = lax.axis_index("c")
    s = lax.axis_index("s")
    wid = c * NS + s
    for i in range(ch // 16):
        obuf[pl.ds(i * 16, 16)] = jnp.ones((16,), jnp.float32)

    @pl.when(s == 0)
    def _():
        pltpu.sync_copy(zn_hbm, deg_sh)

    plsc.subcore_barrier()

    def block(b, carry):
        pltpu.sync_copy(col_hbm.at[wid * nblk + b], call)

        def chunk(k, carry2):
            pltpu.sync_copy(obuf, deg_sh.at[call.at[k]], add=True)
            return carry2

        lax.fori_loop(0, blk, chunk, 0)
        return carry

    lax.fori_loop(0, nblk, block, 0)
    plsc.subcore_barrier()

    @pl.when(s == 0)
    def _():
        pltpu.sync_copy(deg_sh, out_hbm.at[c])


def _make_deg_kernel(n, e, nblk, blk, ch):
    mesh = plsc.VectorSubcoreMesh(core_axis_name="c", subcore_axis_name="s")
    return pl.kernel(
        functools.partial(_deg_body, n, nblk, blk, ch),
        mesh=mesh,
        out_type=jax.ShapeDtypeStruct((NC, n), jnp.float32),
        scratch_types=[
            pltpu.VMEM((blk, ch), jnp.int32),
            pltpu.VMEM((ch,), jnp.float32),
            pltpu.VMEM_SHARED((n,), jnp.float32),
            pltpu.SemaphoreType.DMA,
        ],
    )


def _prop_body(hp, nblk, blk, ch, rps, row_hbm, col_hbm, g_hbm, zg_hbm,
               out_hbm, rall, call, gbuf0, gbuf1, gbuf2, acc_sh, sem0, sem1,
               sem2, ssem0, ssem1, ssem2):
    c = lax.axis_index("c")
    s = lax.axis_index("s")
    wid = c * NS + s
    pltpu.sync_copy(zg_hbm.at[pl.ds(s * rps, rps)], acc_sh.at[pl.ds(s * rps, rps)])
    plsc.subcore_barrier()

    def block(b, carry):
        pltpu.sync_copy(row_hbm.at[wid * nblk + b], rall)
        pltpu.sync_copy(col_hbm.at[wid * nblk + b], call)
        gb = (gbuf0, gbuf1, gbuf2)
        gs = (sem0, sem1, sem2)
        ss = (ssem0, ssem1, ssem2)
        pltpu.async_copy(g_hbm.at[rall.at[0]], gbuf0, sem0)
        pltpu.async_copy(g_hbm.at[rall.at[1]], gbuf1, sem1)

        def one(k, j):
            j2 = (j + 2) % 3
            pltpu.make_async_copy(g_hbm.at[pl.ds(0, ch)], gb[j], gs[j]).wait()
            pltpu.async_copy(gb[j], acc_sh.at[call.at[k]], ss[j], add=True)

            @pl.when(k + 2 < blk)
            def _():
                @pl.when(k >= 1)
                def _():
                    pltpu.make_async_copy(gb[j2], acc_sh.at[pl.ds(0, ch)],
                                          ss[j2]).wait()
                pltpu.async_copy(g_hbm.at[rall.at[k + 2]], gb[j2], gs[j2])

        def triple(i, carry2):
            k0 = i * 3
            one(k0, 0)
            one(k0 + 1, 1)
            one(k0 + 2, 2)
            return carry2

        lax.fori_loop(0, blk // 3, triple, 0)
        for t in range(blk - blk % 3, blk):
            one(t, t % 3)
        for t in (blk - 3, blk - 2, blk - 1):
            pltpu.make_async_copy(gb[t % 3], acc_sh.at[pl.ds(0, ch)],
                                  ss[t % 3]).wait()
        return carry

    lax.fori_loop(0, nblk, block, 0)
    plsc.subcore_barrier()
    pltpu.sync_copy(acc_sh.at[pl.ds(s * rps, rps)],
                    out_hbm.at[c, pl.ds(s * rps, rps)])


def _make_prop_kernel(npad, hp, e, nblk, blk, ch):
    rps = npad // NS
    assert npad % NS == 0 and rps % 8 == 0
    mesh = plsc.VectorSubcoreMesh(core_axis_name="c", subcore_axis_name="s")
    return pl.kernel(
        functools.partial(_prop_body, hp, nblk, blk, ch, rps),
        mesh=mesh,
        out_type=jax.ShapeDtypeStruct((NC, npad, hp), jnp.float32),
        scratch_types=[
            pltpu.VMEM((blk, ch), jnp.int32),
            pltpu.VMEM((blk, ch), jnp.int32),
            pltpu.VMEM((ch, hp), jnp.float32),
            pltpu.VMEM((ch, hp), jnp.float32),
            pltpu.VMEM((ch, hp), jnp.float32),
            pltpu.VMEM_SHARED((npad, hp), jnp.float32),
            pltpu.SemaphoreType.DMA,
            pltpu.SemaphoreType.DMA,
            pltpu.SemaphoreType.DMA,
            pltpu.SemaphoreType.DMA,
            pltpu.SemaphoreType.DMA,
            pltpu.SemaphoreType.DMA,
        ],
    )


def _dinv(degT_ref):
    d = degT_ref[:, 0:1] + degT_ref[:, 1:2] + 1.0
    return lax.rsqrt(d)


def _elu(z):
    return jnp.where(z > 0, z, jnp.exp(z) - 1.0)


def _pad128(g):
    r, hh = g.shape
    return jnp.concatenate([g, jnp.zeros((r, 128 - hh), jnp.float32)], axis=1)


def _k1_body(x_ref, w_ref, degT_ref, p_ref, g_ref):
    dinv = _dinv(degT_ref)
    p = jnp.dot(x_ref[...], w_ref[...], preferred_element_type=jnp.float32)
    p_ref[...] = p
    g_ref[...] = _pad128(p * dinv)


def _k2_body(accp_ref, p_ref, degT_ref, b_ref, w_ref, pn_ref, gn_ref, *, act):
    dinv = _dinv(degT_ref)
    hh64 = p_ref.shape[1]
    acc = (accp_ref[0] + accp_ref[1])[:, :hh64]
    z = dinv * acc + (dinv * dinv) * p_ref[...] + b_ref[...]
    hh = jnp.maximum(z, 0.0) if act == "relu" else _elu(z)
    pn = jnp.dot(hh, w_ref[...], preferred_element_type=jnp.float32)
    pn_ref[...] = pn
    gn_ref[...] = _pad128(pn * dinv)


def _k3_body(accp_ref, p_ref, degT_ref, b_ref, wm_ref, bm_ref, h_ref, s_ref):
    dinv = _dinv(degT_ref)
    hh64 = p_ref.shape[1]
    acc = (accp_ref[0] + accp_ref[1])[:, :hh64]
    z = dinv * acc + (dinv * dinv) * p_ref[...] + b_ref[...]
    hh = _elu(z)
    h_ref[...] = hh
    s_ref[...] = _elu(jnp.dot(hh, wm_ref[...],
                              preferred_element_type=jnp.float32) + bm_ref[...])


def kernel(x, edge_index, dropout, W1, b1, W2, b2, W3, b3, Wm, bm):
    n, d = x.shape
    e = edge_index.shape[1]
    h = W1.shape[1]
    c = Wm.shape[1]
    epw = e // NW
    ch3 = 80
    nch = epw // ch3
    blk = 25
    nblk = nch // blk
    assert e % NW == 0 and epw % ch3 == 0 and nch % blk == 0
    row = edge_index[0].reshape(NW * nblk, blk, ch3)
    col = edge_index[1].reshape(NW * nblk, blk, ch3)

    npad = -(-n // (8 * NS)) * (8 * NS)
    hp = 128
    zn = jnp.zeros((npad,), jnp.float32)
    zg = jnp.zeros((npad, hp), jnp.float32)

    degp = _make_deg_kernel(npad, e, nblk, blk, ch3)(col, zn)
    degT = degp.T

    prop = _make_prop_kernel(npad, hp, e, nblk, blk, ch3)

    R = 2000
    grid = (n // R,)

    def rows(i):
        return (i, 0)

    p1, g1 = pl.pallas_call(
        _k1_body,
        grid=grid,
        in_specs=[
            pl.BlockSpec((R, d), rows),
            pl.BlockSpec((d, h), lambda i: (0, 0)),
            pl.BlockSpec((R, NC), rows),
        ],
        out_specs=[pl.BlockSpec((R, h), rows), pl.BlockSpec((R, hp), rows)],
        out_shape=[jax.ShapeDtypeStruct((n, h), jnp.float32),
                   jax.ShapeDtypeStruct((n, hp), jnp.float32)],
    )(x, W1, degT)

    accp1 = prop(row, col, g1, zg)

    def k2(accp, p, b, w, act):
        return pl.pallas_call(
            functools.partial(_k2_body, act=act),
            grid=grid,
            in_specs=[
                pl.BlockSpec((NC, R, hp), lambda i: (0, i, 0)),
                pl.BlockSpec((R, h), rows),
                pl.BlockSpec((R, NC), rows),
                pl.BlockSpec((1, h), lambda i: (0, 0)),
                pl.BlockSpec((h, h), lambda i: (0, 0)),
            ],
            out_specs=[pl.BlockSpec((R, h), rows), pl.BlockSpec((R, hp), rows)],
            out_shape=[jax.ShapeDtypeStruct((n, h), jnp.float32),
                       jax.ShapeDtypeStruct((n, hp), jnp.float32)],
        )(accp, p, degT, b.reshape(1, h), w)

    p2, g2 = k2(accp1, p1, b1, W2, "relu")
    accp2 = prop(row, col, g2, zg)
    p3, g3 = k2(accp2, p2, b2, W3, "elu")
    accp3 = prop(row, col, g3, zg)

    cp = 128
    wm_pad = jnp.pad(Wm, ((0, 0), (0, cp - c)))
    bm_pad = jnp.pad(bm, (0, cp - c)).reshape(1, cp)

    h3, s_full = pl.pallas_call(
        _k3_body,
        grid=grid,
        in_specs=[
            pl.BlockSpec((NC, R, hp), lambda i: (0, i, 0)),
            pl.BlockSpec((R, h), rows),
            pl.BlockSpec((R, NC), rows),
            pl.BlockSpec((1, h), lambda i: (0, 0)),
            pl.BlockSpec((h, cp), lambda i: (0, 0)),
            pl.BlockSpec((1, cp), lambda i: (0, 0)),
        ],
        out_specs=[pl.BlockSpec((R, h), rows), pl.BlockSpec((R, cp), rows)],
        out_shape=[jax.ShapeDtypeStruct((n, h), jnp.float32),
                   jax.ShapeDtypeStruct((n, cp), jnp.float32)],
    )(accp3, p3, degT, b3.reshape(1, h), wm_pad, bm_pad)

    return (h3, s_full[:, :c])


# speedup vs baseline: 1.0201x; 1.0201x over previous
import functools

import jax
import jax.numpy as jnp
from jax import lax
from jax.experimental import pallas as pl
from jax.experimental.pallas import tpu as pltpu
from jax.experimental.pallas import tpu_sc as plsc

NC = 2
NS = 16
NW = NC * NS


def _deg_body(n, nblk, blk, ch, col_hbm, zn_hbm, out_hbm, call, obuf, deg_sh,
              sem):
    c = lax.axis_index("c")
    s = lax.axis_index("s")
    wid = c * NS + s
    for i in range(ch // 16):
        obuf[pl.ds(i * 16, 16)] = jnp.ones((16,), jnp.float32)

    @pl.when(s == 0)
    def _():
        pltpu.sync_copy(zn_hbm, deg_sh)

    plsc.subcore_barrier()

    def block(b, carry):
        pltpu.sync_copy(col_hbm.at[wid * nblk + b], call)

        def chunk(k, carry2):
            pltpu.sync_copy(obuf, deg_sh.at[call.at[k]], add=True)
            return carry2

        lax.fori_loop(0, blk, chunk, 0)
        return carry

    lax.fori_loop(0, nblk, block, 0)
    plsc.subcore_barrier()

    @pl.when(s == 0)
    def _():
        pltpu.sync_copy(deg_sh, out_hbm.at[c])


def _make_deg_kernel(n, e, nblk, blk, ch):
    mesh = plsc.VectorSubcoreMesh(core_axis_name="c", subcore_axis_name="s")
    return pl.kernel(
        functools.partial(_deg_body, n, nblk, blk, ch),
        mesh=mesh,
        out_type=jax.ShapeDtypeStruct((NC, n), jnp.float32),
        scratch_types=[
            pltpu.VMEM((blk, ch), jnp.int32),
            pltpu.VMEM((ch,), jnp.float32),
            pltpu.VMEM_SHARED((n,), jnp.float32),
            pltpu.SemaphoreType.DMA,
        ],
    )


def _prop_body(hp, nblk, blk, ch, rps, row_hbm, col_hbm, g_hbm, zg_hbm,
               out_hbm, rall, call, gbuf0, gbuf1, gbuf2, gbuf3, acc_sh,
               sem0, sem1, sem2, sem3):
    c = lax.axis_index("c")
    s = lax.axis_index("s")
    wid = c * NS + s
    pltpu.sync_copy(zg_hbm.at[pl.ds(s * rps, rps)], acc_sh.at[pl.ds(s * rps, rps)])
    plsc.subcore_barrier()

    def block(b, carry):
        pltpu.sync_copy(row_hbm.at[wid * nblk + b], rall)
        pltpu.sync_copy(col_hbm.at[wid * nblk + b], call)
        gb = (gbuf0, gbuf1, gbuf2, gbuf3)
        gs = (sem0, sem1, sem2, sem3)
        for j in range(4):
            pltpu.async_copy(g_hbm.at[rall.at[j]], gb[j], gs[j])

        def one(k, j):
            pltpu.make_async_copy(g_hbm.at[pl.ds(0, ch)], gb[j], gs[j]).wait()
            pltpu.sync_copy(gb[j], acc_sh.at[call.at[k]], add=True)

            @pl.when(k + 4 < blk)
            def _():
                pltpu.async_copy(g_hbm.at[rall.at[k + 4]], gb[j], gs[j])

        def quad(i, carry2):
            k0 = i * 4
            for j in range(4):
                one(k0 + j, j)
            return carry2

        lax.fori_loop(0, blk // 4, quad, 0)
        for t in range(blk - blk % 4, blk):
            one(t, t % 4)
        return carry

    lax.fori_loop(0, nblk, block, 0)
    plsc.subcore_barrier()
    pltpu.sync_copy(acc_sh.at[pl.ds(s * rps, rps)],
                    out_hbm.at[c, pl.ds(s * rps, rps)])


def _make_prop_kernel(npad, hp, e, nblk, blk, ch):
    rps = npad // NS
    assert npad % NS == 0 and rps % 8 == 0
    mesh = plsc.VectorSubcoreMesh(core_axis_name="c", subcore_axis_name="s")
    return pl.kernel(
        functools.partial(_prop_body, hp, nblk, blk, ch, rps),
        mesh=mesh,
        out_type=jax.ShapeDtypeStruct((NC, npad, hp), jnp.float32),
        scratch_types=[
            pltpu.VMEM((blk, ch), jnp.int32),
            pltpu.VMEM((blk, ch), jnp.int32),
            pltpu.VMEM((ch, hp), jnp.float32),
            pltpu.VMEM((ch, hp), jnp.float32),
            pltpu.VMEM((ch, hp), jnp.float32),
            pltpu.VMEM((ch, hp), jnp.float32),
            pltpu.VMEM_SHARED((npad, hp), jnp.float32),
            pltpu.SemaphoreType.DMA,
            pltpu.SemaphoreType.DMA,
            pltpu.SemaphoreType.DMA,
            pltpu.SemaphoreType.DMA,
        ],
    )


def _dinv(degT_ref):
    d = degT_ref[:, 0:1] + degT_ref[:, 1:2] + 1.0
    return lax.rsqrt(d)


def _elu(z):
    return jnp.where(z > 0, z, jnp.exp(z) - 1.0)


def _pad128(g):
    r, hh = g.shape
    return jnp.concatenate([g, jnp.zeros((r, 128 - hh), jnp.float32)], axis=1)


def _k1_body(x_ref, w_ref, degT_ref, p_ref, g_ref):
    dinv = _dinv(degT_ref)
    p = jnp.dot(x_ref[...], w_ref[...], preferred_element_type=jnp.float32)
    p_ref[...] = p
    g_ref[...] = _pad128(p * dinv)


def _k2_body(accp_ref, p_ref, degT_ref, b_ref, w_ref, pn_ref, gn_ref, *, act):
    dinv = _dinv(degT_ref)
    hh64 = p_ref.shape[1]
    acc = (accp_ref[0] + accp_ref[1])[:, :hh64]
    z = dinv * acc + (dinv * dinv) * p_ref[...] + b_ref[...]
    hh = jnp.maximum(z, 0.0) if act == "relu" else _elu(z)
    pn = jnp.dot(hh, w_ref[...], preferred_element_type=jnp.float32)
    pn_ref[...] = pn
    gn_ref[...] = _pad128(pn * dinv)


def _k3_body(accp_ref, p_ref, degT_ref, b_ref, wm_ref, bm_ref, h_ref, s_ref):
    dinv = _dinv(degT_ref)
    hh64 = p_ref.shape[1]
    acc = (accp_ref[0] + accp_ref[1])[:, :hh64]
    z = dinv * acc + (dinv * dinv) * p_ref[...] + b_ref[...]
    hh = _elu(z)
    h_ref[...] = hh
    s_ref[...] = _elu(jnp.dot(hh, wm_ref[...],
                              preferred_element_type=jnp.float32) + bm_ref[...])


def kernel(x, edge_index, dropout, W1, b1, W2, b2, W3, b3, Wm, bm):
    n, d = x.shape
    e = edge_index.shape[1]
    h = W1.shape[1]
    c = Wm.shape[1]
    epw = e // NW
    ch3 = 80
    nch = epw // ch3
    blk = 25
    nblk = nch // blk
    assert e % NW == 0 and epw % ch3 == 0 and nch % blk == 0
    row = edge_index[0].reshape(NW * nblk, blk, ch3)
    col = edge_index[1].reshape(NW * nblk, blk, ch3)

    npad = -(-n // (8 * NS)) * (8 * NS)
    hp = 128
    zn = jnp.zeros((npad,), jnp.float32)
    zg = jnp.zeros((npad, hp), jnp.float32)

    degp = _make_deg_kernel(npad, e, nblk, blk, ch3)(col, zn)
    degT = degp.T

    prop = _make_prop_kernel(npad, hp, e, nblk, blk, ch3)

    R = 2000
    grid = (n // R,)

    def rows(i):
        return (i, 0)

    p1, g1 = pl.pallas_call(
        _k1_body,
        grid=grid,
        in_specs=[
            pl.BlockSpec((R, d), rows),
            pl.BlockSpec((d, h), lambda i: (0, 0)),
            pl.BlockSpec((R, NC), rows),
        ],
        out_specs=[pl.BlockSpec((R, h), rows), pl.BlockSpec((R, hp), rows)],
        out_shape=[jax.ShapeDtypeStruct((n, h), jnp.float32),
                   jax.ShapeDtypeStruct((n, hp), jnp.float32)],
    )(x, W1, degT)

    accp1 = prop(row, col, g1, zg)

    def k2(accp, p, b, w, act):
        return pl.pallas_call(
            functools.partial(_k2_body, act=act),
            grid=grid,
            in_specs=[
                pl.BlockSpec((NC, R, hp), lambda i: (0, i, 0)),
                pl.BlockSpec((R, h), rows),
                pl.BlockSpec((R, NC), rows),
                pl.BlockSpec((1, h), lambda i: (0, 0)),
                pl.BlockSpec((h, h), lambda i: (0, 0)),
            ],
            out_specs=[pl.BlockSpec((R, h), rows), pl.BlockSpec((R, hp), rows)],
            out_shape=[jax.ShapeDtypeStruct((n, h), jnp.float32),
                       jax.ShapeDtypeStruct((n, hp), jnp.float32)],
        )(accp, p, degT, b.reshape(1, h), w)

    p2, g2 = k2(accp1, p1, b1, W2, "relu")
    accp2 = prop(row, col, g2, zg)
    p3, g3 = k2(accp2, p2, b2, W3, "elu")
    accp3 = prop(row, col, g3, zg)

    cp = 128
    wm_pad = jnp.pad(Wm, ((0, 0), (0, cp - c)))
    bm_pad = jnp.pad(bm, (0, cp - c)).reshape(1, cp)

    h3, s_full = pl.pallas_call(
        _k3_body,
        grid=grid,
        in_specs=[
            pl.BlockSpec((NC, R, hp), lambda i: (0, i, 0)),
            pl.BlockSpec((R, h), rows),
            pl.BlockSpec((R, NC), rows),
            pl.BlockSpec((1, h), lambda i: (0, 0)),
            pl.BlockSpec((h, cp), lambda i: (0, 0)),
            pl.BlockSpec((1, cp), lambda i: (0, 0)),
        ],
        out_specs=[pl.BlockSpec((R, h), rows), pl.BlockSpec((R, cp), rows)],
        out_shape=[jax.ShapeDtypeStruct((n, h), jnp.float32),
                   jax.ShapeDtypeStruct((n, cp), jnp.float32)],
    )(accp3, p3, degT, b3.reshape(1, h), wm_pad, bm_pad)

    return (h3, s_full[:, :c])
